# baseline (speedup 1.0000x reference)
import functools
import jax
import jax.numpy as jnp
from jax import lax
from jax.experimental import pallas as pl
from jax.experimental.pallas import tpu as pltpu, tpu_sc as plsc

G = 128
H = 64
C_IN = 128
C_OUT = 128
EPS = 1e-5

BN = 512
N_TOTAL = 2 * 16384
NPAD = N_TOTAL + BN

_info = plsc.get_sparse_core_info()
_NC = _info.num_cores
_NS = _info.num_subcores
_NW = _NC * _NS
N_PER_W = N_TOTAL // _NW
CHUNK = 128
N_CHUNKS = N_PER_W // CHUNK


def _lift_body(p_ref, w_ref, b_ref, o_ref):
    o_ref[...] = (
        jnp.dot(p_ref[...], w_ref[...], preferred_element_type=jnp.float32)
        + b_ref[...]
    )


def _lift(points, w1, b1):
    n = points.shape[0]
    return pl.pallas_call(
        _lift_body,
        grid=(n // BN,),
        in_specs=[
            pl.BlockSpec((BN, C_IN), lambda i: (i, 0)),
            pl.BlockSpec((C_IN, H), lambda i: (0, 0)),
            pl.BlockSpec((1, H), lambda i: (0, 0)),
        ],
        out_specs=pl.BlockSpec((BN, H), lambda i: (i, 0)),
        out_shape=jax.ShapeDtypeStruct((n, H), jnp.float32),
    )(points, w1, b1.reshape(1, H))


def _ymm_body(f_ref, w_ref, y_ref):
    y = jnp.dot(f_ref[...], w_ref[0], preferred_element_type=jnp.float32)
    y_ref[...] = jnp.pad(y, ((0, 0), (0, H)))[None]


def _ymm(featsp, cw):
    return pl.pallas_call(
        _ymm_body,
        grid=(NPAD // BN, 27),
        in_specs=[
            pl.BlockSpec((BN, H), lambda i, o: (i, 0)),
            pl.BlockSpec((1, H, H), lambda i, o: (o, 0, 0)),
        ],
        out_specs=pl.BlockSpec((1, BN, 2 * H), lambda i, o: (o, i, 0)),
        out_shape=jax.ShapeDtypeStruct((27, NPAD, 2 * H), jnp.float32),
    )(featsp, cw)


def _sc_body(yz_hbm, gidx_hbm, feats_hbm, cb_hbm, out_hbm,
             idx_v, buf_v, acc_v, cb_v, sem):
    wid = lax.axis_index("s") * _NC + lax.axis_index("c")
    base = wid * N_PER_W
    pltpu.sync_copy(gidx_hbm.at[wid], idx_v)
    pltpu.sync_copy(cb_hbm, cb_v)

    def chunk_body(c, _):
        row0 = base + c * CHUNK
        pltpu.sync_copy(feats_hbm.at[pl.ds(row0, CHUNK)], acc_v)

        def add_cb(r, _):
            for j in range(H // 16):
                sl = pl.ds(j * 16, 16)
                acc_v[r, sl] = acc_v[r, sl] + cb_v[sl]
            return 0

        lax.fori_loop(0, CHUNK, add_cb, 0)

        def off_body(o, _):
            cp = pltpu.async_copy(
                yz_hbm.at[idx_v.at[o, pl.ds(c * CHUNK, CHUNK)]], buf_v, sem
            )
            cp.wait()

            def row_body(r, _):
                for j in range(H // 16):
                    sl = pl.ds(j * 16, 16)
                    acc_v[r, sl] = acc_v[r, sl] + buf_v[r, sl]
                return 0

            lax.fori_loop(0, CHUNK, row_body, 0)
            return 0

        lax.fori_loop(0, 27, off_body, 0)
        pltpu.sync_copy(acc_v, out_hbm.at[pl.ds(row0, CHUNK)])
        return 0

    lax.fori_loop(0, N_CHUNKS, chunk_body, 0)


def _sc_gather_sum(yz, gidx_w, feats, cb):
    mesh = plsc.VectorSubcoreMesh(core_axis_name="c", subcore_axis_name="s")
    f = functools.partial(
        pl.kernel,
        mesh=mesh,
        out_type=jax.ShapeDtypeStruct((N_TOTAL, H), jnp.float32),
        scratch_types=[
            pltpu.VMEM((27, N_PER_W), jnp.int32),
            pltpu.VMEM((CHUNK, 2 * H), jnp.float32),
            pltpu.VMEM((CHUNK, H), jnp.float32),
            pltpu.VMEM((H,), jnp.float32),
            pltpu.SemaphoreType.DMA,
        ],
    )(_sc_body)
    return f(yz, gidx_w, feats, cb)


def _final_body(f_ref, p_ref, w2_ref, b2_ref, g_ref, be_ref, w3_ref,
                b3_ref, o_ref):
    t = jnp.dot(f_ref[...], w2_ref[...], preferred_element_type=jnp.float32)
    t = jnp.maximum(t + b2_ref[...], 0.0)
    mu = jnp.mean(t, axis=-1, keepdims=True)
    var = jnp.mean((t - mu) ** 2, axis=-1, keepdims=True)
    t = (t - mu) * lax.rsqrt(var + EPS) * g_ref[...] + be_ref[...]
    t = t + p_ref[...]
    o_ref[...] = (
        jnp.dot(t, w3_ref[...], preferred_element_type=jnp.float32)
        + b3_ref[...]
    )


def _final(feats, points, w2, b2, gamma, beta, w3, b3):
    n = feats.shape[0]
    return pl.pallas_call(
        _final_body,
        grid=(n // BN,),
        in_specs=[
            pl.BlockSpec((BN, H), lambda i: (i, 0)),
            pl.BlockSpec((BN, C_OUT), lambda i: (i, 0)),
            pl.BlockSpec((H, C_OUT), lambda i: (0, 0)),
            pl.BlockSpec((1, C_OUT), lambda i: (0, 0)),
            pl.BlockSpec((1, C_OUT), lambda i: (0, 0)),
            pl.BlockSpec((1, C_OUT), lambda i: (0, 0)),
            pl.BlockSpec((C_OUT, C_OUT), lambda i: (0, 0)),
            pl.BlockSpec((1, C_OUT), lambda i: (0, 0)),
        ],
        out_specs=pl.BlockSpec((BN, C_OUT), lambda i: (i, 0)),
        out_shape=jax.ShapeDtypeStruct((n, C_OUT), jnp.float32),
    )(feats, points, w2, b2.reshape(1, -1), gamma.reshape(1, -1),
      beta.reshape(1, -1), w3, b3.reshape(1, -1))


@jax.jit
def kernel(p, x, w1, b1, cw1, cb1, cw2, cb2, w2, b2, gamma, beta, w3, b3):
    b_, c, n = x.shape
    N = b_ * n
    xyz = jnp.transpose(p, (0, 2, 1))
    points = jnp.transpose(x, (0, 2, 1)).reshape(N, C_IN)

    norm_p = (xyz + 1.0) / 2.0
    idx = jnp.clip((norm_p * (G - 1)).astype(jnp.int32), 0, G - 1)
    idx_f = idx.reshape(-1, 3)
    batch_ids = jnp.repeat(jnp.arange(b_, dtype=jnp.int32), n)
    keys = ((batch_ids * G + idx_f[:, 0]) * G + idx_f[:, 1]) * G + idx_f[:, 2]
    order = jnp.argsort(keys)
    sorted_keys = keys[order]
    offs = jnp.array(
        [(dx, dy, dz) for dx in (-1, 0, 1) for dy in (-1, 0, 1)
         for dz in (-1, 0, 1)], dtype=jnp.int32)
    nbr = idx_f[None, :, :] + offs[:, None, :]
    valid = jnp.all((nbr >= 0) & (nbr <= G - 1), axis=-1)
    nbr_c = jnp.clip(nbr, 0, G - 1)
    qkeys = ((batch_ids[None, :] * G + nbr_c[..., 0]) * G
             + nbr_c[..., 1]) * G + nbr_c[..., 2]
    pos = jnp.searchsorted(sorted_keys, qkeys)
    pos_c = jnp.clip(pos, 0, N - 1)
    found = (sorted_keys[pos_c] == qkeys) & valid
    rows = order[pos_c].astype(jnp.int32)

    o_base = (jnp.arange(27, dtype=jnp.int32) * NPAD)[:, None]
    gidx = jnp.where(found, o_base + rows, o_base + N)
    gidx_w = gidx.reshape(27, _NW, N_PER_W).transpose(1, 0, 2)

    feats = _lift(points, w1, b1)

    for cw, cb in ((cw1, cb1), (cw2, cb2)):
        featsp = jnp.pad(feats, ((0, NPAD - N), (0, 0)))
        yz = _ymm(featsp, cw).reshape(27 * NPAD, 2 * H)
        feats = _sc_gather_sum(yz, gidx_w, feats, cb)

    out = _final(feats, points, w2, b2, gamma, beta, w3, b3)
    return out.reshape(b_, n, C_OUT)


# speedup vs baseline: 23.7703x; 23.7703x over previous
import functools
import jax
import jax.numpy as jnp
from jax import lax
from jax.experimental import pallas as pl
from jax.experimental.pallas import tpu as pltpu, tpu_sc as plsc

G = 128
H = 64
C_IN = 128
C_OUT = 128
EPS = 1e-5

BN = 512
N_TOTAL = 2 * 16384
NPAD = N_TOTAL + BN

_info = plsc.get_sparse_core_info()
_NC = _info.num_cores
_NS = _info.num_subcores
_NW = _NC * _NS
N_PER_W = N_TOTAL // _NW
CHUNK = 128
N_CHUNKS = N_PER_W // CHUNK


def _lift_body(p_ref, w_ref, b_ref, o_ref):
    o_ref[...] = (
        jnp.dot(p_ref[...], w_ref[...], preferred_element_type=jnp.float32)
        + b_ref[...]
    )


def _lift(points, w1, b1):
    n = points.shape[0]
    return pl.pallas_call(
        _lift_body,
        grid=(n // BN,),
        in_specs=[
            pl.BlockSpec((BN, C_IN), lambda i: (i, 0)),
            pl.BlockSpec((C_IN, H), lambda i: (0, 0)),
            pl.BlockSpec((1, H), lambda i: (0, 0)),
        ],
        out_specs=pl.BlockSpec((BN, H), lambda i: (i, 0)),
        out_shape=jax.ShapeDtypeStruct((n, H), jnp.float32),
    )(points, w1, b1.reshape(1, H))


def _ymm_body(f_ref, w_ref, y_ref):
    y = jnp.dot(f_ref[...], w_ref[0], preferred_element_type=jnp.float32)
    y_ref[...] = jnp.pad(y, ((0, 0), (0, H)))[None]


def _ymm(featsp, cw):
    return pl.pallas_call(
        _ymm_body,
        grid=(NPAD // BN, 27),
        in_specs=[
            pl.BlockSpec((BN, H), lambda i, o: (i, 0)),
            pl.BlockSpec((1, H, H), lambda i, o: (o, 0, 0)),
        ],
        out_specs=pl.BlockSpec((1, BN, 2 * H), lambda i, o: (o, i, 0)),
        out_shape=jax.ShapeDtypeStruct((27, NPAD, 2 * H), jnp.float32),
    )(featsp, cw)


def _sc_body(yz_hbm, gidx_hbm, feats_hbm, cb_hbm, out_hbm,
             idx_v, buf_v, acc_v, cb_v, sem):
    wid = lax.axis_index("s") * _NC + lax.axis_index("c")
    base = wid * N_PER_W
    pltpu.sync_copy(gidx_hbm.at[wid], idx_v)
    pltpu.sync_copy(cb_hbm, cb_v)

    def chunk_body(c, _):
        row0 = base + c * CHUNK
        pltpu.sync_copy(feats_hbm.at[pl.ds(row0, CHUNK)], acc_v)

        def add_cb(r, _):
            for j in range(H // 16):
                sl = pl.ds(j * 16, 16)
                acc_v[r, sl] = acc_v[r, sl] + cb_v[sl]
            return 0

        lax.fori_loop(0, CHUNK, add_cb, 0)

        def off_body(o, _):
            cp = pltpu.async_copy(
                yz_hbm.at[idx_v.at[o, pl.ds(c * CHUNK, CHUNK)]], buf_v, sem
            )
            cp.wait()

            def row_body(r, _):
                for j in range(H // 16):
                    sl = pl.ds(j * 16, 16)
                    acc_v[r, sl] = acc_v[r, sl] + buf_v[r, sl]
                return 0

            lax.fori_loop(0, CHUNK, row_body, 0)
            return 0

        lax.fori_loop(0, 27, off_body, 0)
        pltpu.sync_copy(acc_v, out_hbm.at[pl.ds(row0, CHUNK)])
        return 0

    lax.fori_loop(0, N_CHUNKS, chunk_body, 0)


def _sc_gather_sum(yz, gidx_w, feats, cb):
    mesh = plsc.VectorSubcoreMesh(core_axis_name="c", subcore_axis_name="s")
    f = functools.partial(
        pl.kernel,
        mesh=mesh,
        out_type=jax.ShapeDtypeStruct((N_TOTAL, H), jnp.float32),
        scratch_types=[
            pltpu.VMEM((27, N_PER_W), jnp.int32),
            pltpu.VMEM((CHUNK, 2 * H), jnp.float32),
            pltpu.VMEM((CHUNK, H), jnp.float32),
            pltpu.VMEM((H,), jnp.float32),
            pltpu.SemaphoreType.DMA,
        ],
    )(_sc_body)
    return f(yz, gidx_w, feats, cb)


def _final_body(f_ref, p_ref, w2_ref, b2_ref, g_ref, be_ref, w3_ref,
                b3_ref, o_ref):
    t = jnp.dot(f_ref[...], w2_ref[...], preferred_element_type=jnp.float32)
    t = jnp.maximum(t + b2_ref[...], 0.0)
    mu = jnp.mean(t, axis=-1, keepdims=True)
    var = jnp.mean((t - mu) ** 2, axis=-1, keepdims=True)
    t = (t - mu) * lax.rsqrt(var + EPS) * g_ref[...] + be_ref[...]
    t = t + p_ref[...]
    o_ref[...] = (
        jnp.dot(t, w3_ref[...], preferred_element_type=jnp.float32)
        + b3_ref[...]
    )


def _final(feats, points, w2, b2, gamma, beta, w3, b3):
    n = feats.shape[0]
    return pl.pallas_call(
        _final_body,
        grid=(n // BN,),
        in_specs=[
            pl.BlockSpec((BN, H), lambda i: (i, 0)),
            pl.BlockSpec((BN, C_OUT), lambda i: (i, 0)),
            pl.BlockSpec((H, C_OUT), lambda i: (0, 0)),
            pl.BlockSpec((1, C_OUT), lambda i: (0, 0)),
            pl.BlockSpec((1, C_OUT), lambda i: (0, 0)),
            pl.BlockSpec((1, C_OUT), lambda i: (0, 0)),
            pl.BlockSpec((C_OUT, C_OUT), lambda i: (0, 0)),
            pl.BlockSpec((1, C_OUT), lambda i: (0, 0)),
        ],
        out_specs=pl.BlockSpec((BN, C_OUT), lambda i: (i, 0)),
        out_shape=jax.ShapeDtypeStruct((n, C_OUT), jnp.float32),
    )(feats, points, w2, b2.reshape(1, -1), gamma.reshape(1, -1),
      beta.reshape(1, -1), w3, b3.reshape(1, -1))


@jax.jit
def kernel(p, x, w1, b1, cw1, cb1, cw2, cb2, w2, b2, gamma, beta, w3, b3):
    b_, c, n = x.shape
    N = b_ * n
    xyz = jnp.transpose(p, (0, 2, 1))
    points = jnp.transpose(x, (0, 2, 1)).reshape(N, C_IN)

    norm_p = (xyz + 1.0) / 2.0
    idx = jnp.clip((norm_p * (G - 1)).astype(jnp.int32), 0, G - 1)
    idx_f = idx.reshape(-1, 3)
    batch_ids = jnp.repeat(jnp.arange(b_, dtype=jnp.int32), n)
    keys = ((batch_ids * G + idx_f[:, 0]) * G + idx_f[:, 1]) * G + idx_f[:, 2]
    offs = jnp.array(
        [(dx, dy, dz) for dx in (-1, 0, 1) for dy in (-1, 0, 1)
         for dz in (-1, 0, 1)], dtype=jnp.int32)
    nbr = idx_f[None, :, :] + offs[:, None, :]
    valid = jnp.all((nbr >= 0) & (nbr <= G - 1), axis=-1)
    nbr_c = jnp.clip(nbr, 0, G - 1)
    qkeys = ((batch_ids[None, :] * G + nbr_c[..., 0]) * G
             + nbr_c[..., 1]) * G + nbr_c[..., 2]
    tab = jnp.full((b_ * G * G * G,), N, jnp.int32)
    tab = tab.at[keys].min(jnp.arange(N, dtype=jnp.int32))
    tv = tab[jnp.where(valid, qkeys, 0)]
    found = valid & (tv < N)
    o_base = (jnp.arange(27, dtype=jnp.int32) * NPAD)[:, None]
    gidx = jnp.where(found, o_base + tv, o_base + N)
    gidx_w = gidx.reshape(27, _NW, N_PER_W).transpose(1, 0, 2)

    feats = _lift(points, w1, b1)

    for cw, cb in ((cw1, cb1), (cw2, cb2)):
        featsp = jnp.pad(feats, ((0, NPAD - N), (0, 0)))
        yz = _ymm(featsp, cw).reshape(27 * NPAD, 2 * H)
        feats = _sc_gather_sum(yz, gidx_w, feats, cb)

    out = _final(feats, points, w2, b2, gamma, beta, w3, b3)
    return out.reshape(b_, n, C_OUT)


# speedup vs baseline: 27.2594x; 1.1468x over previous
import functools
import jax
import jax.numpy as jnp
from jax import lax
from jax.experimental import pallas as pl
from jax.experimental.pallas import tpu as pltpu, tpu_sc as plsc

G = 128
H = 64
C_IN = 128
C_OUT = 128
EPS = 1e-5

BN = 512
N_TOTAL = 2 * 16384
NPAD = N_TOTAL + BN

_info = plsc.get_sparse_core_info()
_NC = _info.num_cores
_NS = _info.num_subcores
_NW = _NC * _NS
N_PER_W = N_TOTAL // _NW
CHUNK = 32
N_CHUNKS = N_PER_W // CHUNK


def _lift_body(p_ref, w_ref, b_ref, o_ref):
    o_ref[...] = (
        jnp.dot(p_ref[...], w_ref[...], preferred_element_type=jnp.float32)
        + b_ref[...]
    )


def _lift(points, w1, b1):
    n = points.shape[0]
    return pl.pallas_call(
        _lift_body,
        grid=(n // BN,),
        in_specs=[
            pl.BlockSpec((BN, C_IN), lambda i: (i, 0)),
            pl.BlockSpec((C_IN, H), lambda i: (0, 0)),
            pl.BlockSpec((1, H), lambda i: (0, 0)),
        ],
        out_specs=pl.BlockSpec((BN, H), lambda i: (i, 0)),
        out_shape=jax.ShapeDtypeStruct((n, H), jnp.float32),
    )(points, w1, b1.reshape(1, H))


def _ymm_body(f_ref, w_ref, y_ref):
    y = jnp.dot(f_ref[...], w_ref[0], preferred_element_type=jnp.float32)
    y_ref[...] = jnp.pad(y, ((0, 0), (0, H)))[None]


def _ymm(featsp, cw):
    return pl.pallas_call(
        _ymm_body,
        grid=(NPAD // BN, 27),
        in_specs=[
            pl.BlockSpec((BN, H), lambda i, o: (i, 0)),
            pl.BlockSpec((1, H, H), lambda i, o: (o, 0, 0)),
        ],
        out_specs=pl.BlockSpec((1, BN, 2 * H), lambda i, o: (o, i, 0)),
        out_shape=jax.ShapeDtypeStruct((27, NPAD, 2 * H), jnp.float32),
    )(featsp, cw)


def _sc_body(yz_hbm, gidx_hbm, feats_hbm, cb_hbm, out_hbm,
             idx_v, big_v, acc_v, cb_v, sem):
    wid = lax.axis_index("s") * _NC + lax.axis_index("c")
    base = wid * N_PER_W
    pltpu.sync_copy(cb_hbm, cb_v)

    def chunk_body(c, _):
        row0 = base + c * CHUNK
        pltpu.sync_copy(gidx_hbm.at[wid, c], idx_v)
        pltpu.sync_copy(feats_hbm.at[pl.ds(row0, CHUNK)], acc_v)

        def fire(o, _):
            pltpu.async_copy(
                yz_hbm.at[idx_v.at[o]],
                big_v.at[pl.ds(o * CHUNK, CHUNK)],
                sem,
            )
            return 0

        lax.fori_loop(0, 27, fire, 0)

        def add_cb(r, _):
            for j in range(H // 16):
                sl = pl.ds(j * 16, 16)
                acc_v[r, sl] = acc_v[r, sl] + cb_v[sl]
            return 0

        lax.fori_loop(0, CHUNK, add_cb, 0)

        def drain(o, _):
            pltpu.make_async_copy(
                yz_hbm.at[idx_v.at[o]],
                big_v.at[pl.ds(o * CHUNK, CHUNK)],
                sem,
            ).wait()

            def row_body(r, _):
                for j in range(H // 16):
                    sl = pl.ds(j * 16, 16)
                    acc_v[r, sl] = acc_v[r, sl] + big_v[o * CHUNK + r, sl]
                return 0

            lax.fori_loop(0, CHUNK, row_body, 0)
            return 0

        lax.fori_loop(0, 27, drain, 0)
        pltpu.sync_copy(acc_v, out_hbm.at[pl.ds(row0, CHUNK)])
        return 0

    lax.fori_loop(0, N_CHUNKS, chunk_body, 0)


def _sc_gather_sum(yz, gidx_w, feats, cb):
    mesh = plsc.VectorSubcoreMesh(core_axis_name="c", subcore_axis_name="s")
    f = functools.partial(
        pl.kernel,
        mesh=mesh,
        out_type=jax.ShapeDtypeStruct((N_TOTAL, H), jnp.float32),
        scratch_types=[
            pltpu.VMEM((27, CHUNK), jnp.int32),
            pltpu.VMEM((27 * CHUNK, 2 * H), jnp.float32),
            pltpu.VMEM((CHUNK, H), jnp.float32),
            pltpu.VMEM((H,), jnp.float32),
            pltpu.SemaphoreType.DMA,
        ],
    )(_sc_body)
    return f(yz, gidx_w, feats, cb)


def _final_body(f_ref, p_ref, w2_ref, b2_ref, g_ref, be_ref, w3_ref,
                b3_ref, o_ref):
    t = jnp.dot(f_ref[...], w2_ref[...], preferred_element_type=jnp.float32)
    t = jnp.maximum(t + b2_ref[...], 0.0)
    mu = jnp.mean(t, axis=-1, keepdims=True)
    var = jnp.mean((t - mu) ** 2, axis=-1, keepdims=True)
    t = (t - mu) * lax.rsqrt(var + EPS) * g_ref[...] + be_ref[...]
    t = t + p_ref[...]
    o_ref[...] = (
        jnp.dot(t, w3_ref[...], preferred_element_type=jnp.float32)
        + b3_ref[...]
    )


def _final(feats, points, w2, b2, gamma, beta, w3, b3):
    n = feats.shape[0]
    return pl.pallas_call(
        _final_body,
        grid=(n // BN,),
        in_specs=[
            pl.BlockSpec((BN, H), lambda i: (i, 0)),
            pl.BlockSpec((BN, C_OUT), lambda i: (i, 0)),
            pl.BlockSpec((H, C_OUT), lambda i: (0, 0)),
            pl.BlockSpec((1, C_OUT), lambda i: (0, 0)),
            pl.BlockSpec((1, C_OUT), lambda i: (0, 0)),
            pl.BlockSpec((1, C_OUT), lambda i: (0, 0)),
            pl.BlockSpec((C_OUT, C_OUT), lambda i: (0, 0)),
            pl.BlockSpec((1, C_OUT), lambda i: (0, 0)),
        ],
        out_specs=pl.BlockSpec((BN, C_OUT), lambda i: (i, 0)),
        out_shape=jax.ShapeDtypeStruct((n, C_OUT), jnp.float32),
    )(feats, points, w2, b2.reshape(1, -1), gamma.reshape(1, -1),
      beta.reshape(1, -1), w3, b3.reshape(1, -1))


@jax.jit
def kernel(p, x, w1, b1, cw1, cb1, cw2, cb2, w2, b2, gamma, beta, w3, b3):
    b_, c, n = x.shape
    N = b_ * n
    xyz = jnp.transpose(p, (0, 2, 1))
    points = jnp.transpose(x, (0, 2, 1)).reshape(N, C_IN)

    norm_p = (xyz + 1.0) / 2.0
    idx = jnp.clip((norm_p * (G - 1)).astype(jnp.int32), 0, G - 1)
    idx_f = idx.reshape(-1, 3)
    batch_ids = jnp.repeat(jnp.arange(b_, dtype=jnp.int32), n)
    keys = ((batch_ids * G + idx_f[:, 0]) * G + idx_f[:, 1]) * G + idx_f[:, 2]
    offs = jnp.array(
        [(dx, dy, dz) for dx in (-1, 0, 1) for dy in (-1, 0, 1)
         for dz in (-1, 0, 1)], dtype=jnp.int32)
    nbr = idx_f[None, :, :] + offs[:, None, :]
    valid = jnp.all((nbr >= 0) & (nbr <= G - 1), axis=-1)
    nbr_c = jnp.clip(nbr, 0, G - 1)
    qkeys = ((batch_ids[None, :] * G + nbr_c[..., 0]) * G
             + nbr_c[..., 1]) * G + nbr_c[..., 2]
    tab = jnp.full((b_ * G * G * G,), N, jnp.int32)
    tab = tab.at[keys].min(jnp.arange(N, dtype=jnp.int32))
    tv = tab[jnp.where(valid, qkeys, 0)]
    found = valid & (tv < N)
    o_base = (jnp.arange(27, dtype=jnp.int32) * NPAD)[:, None]
    gidx = jnp.where(found, o_base + tv, o_base + N)
    gidx_w = gidx.reshape(27, _NW, N_CHUNKS, CHUNK).transpose(1, 2, 0, 3)

    feats = _lift(points, w1, b1)

    for cw, cb in ((cw1, cb1), (cw2, cb2)):
        featsp = jnp.pad(feats, ((0, NPAD - N), (0, 0)))
        yz = _ymm(featsp, cw).reshape(27 * NPAD, 2 * H)
        feats = _sc_gather_sum(yz, gidx_w, feats, cb)

    out = _final(feats, points, w2, b2, gamma, beta, w3, b3)
    return out.reshape(b_, n, C_OUT)


# speedup vs baseline: 27.3282x; 1.0025x over previous
import functools
import jax
import jax.numpy as jnp
from jax import lax
from jax.experimental import pallas as pl
from jax.experimental.pallas import tpu as pltpu, tpu_sc as plsc

G = 128
H = 64
C_IN = 128
C_OUT = 128
EPS = 1e-5

BN = 512
N_TOTAL = 2 * 16384
NPAD = N_TOTAL + BN

_info = plsc.get_sparse_core_info()
_NC = _info.num_cores
_NS = _info.num_subcores
_NW = _NC * _NS
N_PER_W = N_TOTAL // _NW
CHUNK = 32
N_CHUNKS = N_PER_W // CHUNK


def _lift_body(p_ref, w_ref, b_ref, o_ref):
    o_ref[...] = (
        jnp.dot(p_ref[...], w_ref[...], preferred_element_type=jnp.float32)
        + b_ref[...]
    )


def _lift(points, w1, b1):
    n = points.shape[0]
    return pl.pallas_call(
        _lift_body,
        grid=(n // BN,),
        in_specs=[
            pl.BlockSpec((BN, C_IN), lambda i: (i, 0)),
            pl.BlockSpec((C_IN, H), lambda i: (0, 0)),
            pl.BlockSpec((1, H), lambda i: (0, 0)),
        ],
        out_specs=pl.BlockSpec((BN, H), lambda i: (i, 0)),
        out_shape=jax.ShapeDtypeStruct((n, H), jnp.float32),
    )(points, w1, b1.reshape(1, H))


def _ymm_body(f_ref, w_ref, y_ref):
    y = jnp.dot(f_ref[...], w_ref[0], preferred_element_type=jnp.float32)
    y_ref[...] = jnp.pad(y, ((0, 0), (0, H)))[None]


def _ymm(featsp, cw):
    return pl.pallas_call(
        _ymm_body,
        grid=(NPAD // BN, 27),
        in_specs=[
            pl.BlockSpec((BN, H), lambda i, o: (i, 0)),
            pl.BlockSpec((1, H, H), lambda i, o: (o, 0, 0)),
        ],
        out_specs=pl.BlockSpec((1, BN, 2 * H), lambda i, o: (o, i, 0)),
        out_shape=jax.ShapeDtypeStruct((27, NPAD, 2 * H), jnp.float32),
    )(featsp, cw)


def _sc_body(yz_hbm, gidx_hbm, feats_hbm, cb_hbm, out_hbm,
             idx_v, big_v, acc_v, cb_v, sem):
    wid = lax.axis_index("s") * _NC + lax.axis_index("c")
    base = wid * N_PER_W
    pltpu.sync_copy(cb_hbm, cb_v)

    def chunk_body(c, _):
        row0 = base + c * CHUNK
        pltpu.sync_copy(gidx_hbm.at[wid, c], idx_v)
        pltpu.sync_copy(feats_hbm.at[pl.ds(row0, CHUNK)], acc_v)

        def fire(o, _):
            pltpu.async_copy(
                yz_hbm.at[idx_v.at[o]],
                big_v.at[pl.ds(o * CHUNK, CHUNK)],
                sem,
            )
            return 0

        lax.fori_loop(0, 27, fire, 0)

        def drain(o, _):
            pltpu.make_async_copy(
                yz_hbm.at[idx_v.at[o]],
                big_v.at[pl.ds(o * CHUNK, CHUNK)],
                sem,
            ).wait()
            return 0

        lax.fori_loop(0, 27, drain, 0)

        sls = [pl.ds(j * 16, 16) for j in range(H // 16)]

        def row_body(r, _):
            init = tuple(acc_v[r, sl] + cb_v[sl] for sl in sls)

            def o_body(o, carry):
                return tuple(
                    a + big_v[o * CHUNK + r, sl]
                    for a, sl in zip(carry, sls)
                )

            res = lax.fori_loop(0, 27, o_body, init)
            for a, sl in zip(res, sls):
                acc_v[r, sl] = a
            return 0

        lax.fori_loop(0, CHUNK, row_body, 0)
        pltpu.sync_copy(acc_v, out_hbm.at[pl.ds(row0, CHUNK)])
        return 0

    lax.fori_loop(0, N_CHUNKS, chunk_body, 0)


def _sc_gather_sum(yz, gidx_w, feats, cb):
    mesh = plsc.VectorSubcoreMesh(core_axis_name="c", subcore_axis_name="s")
    f = functools.partial(
        pl.kernel,
        mesh=mesh,
        out_type=jax.ShapeDtypeStruct((N_TOTAL, H), jnp.float32),
        scratch_types=[
            pltpu.VMEM((27, CHUNK), jnp.int32),
            pltpu.VMEM((27 * CHUNK, 2 * H), jnp.float32),
            pltpu.VMEM((CHUNK, H), jnp.float32),
            pltpu.VMEM((H,), jnp.float32),
            pltpu.SemaphoreType.DMA,
        ],
    )(_sc_body)
    return f(yz, gidx_w, feats, cb)


def _final_body(f_ref, p_ref, w2_ref, b2_ref, g_ref, be_ref, w3_ref,
                b3_ref, o_ref):
    t = jnp.dot(f_ref[...], w2_ref[...], preferred_element_type=jnp.float32)
    t = jnp.maximum(t + b2_ref[...], 0.0)
    mu = jnp.mean(t, axis=-1, keepdims=True)
    var = jnp.mean((t - mu) ** 2, axis=-1, keepdims=True)
    t = (t - mu) * lax.rsqrt(var + EPS) * g_ref[...] + be_ref[...]
    t = t + p_ref[...]
    o_ref[...] = (
        jnp.dot(t, w3_ref[...], preferred_element_type=jnp.float32)
        + b3_ref[...]
    )


def _final(feats, points, w2, b2, gamma, beta, w3, b3):
    n = feats.shape[0]
    return pl.pallas_call(
        _final_body,
        grid=(n // BN,),
        in_specs=[
            pl.BlockSpec((BN, H), lambda i: (i, 0)),
            pl.BlockSpec((BN, C_OUT), lambda i: (i, 0)),
            pl.BlockSpec((H, C_OUT), lambda i: (0, 0)),
            pl.BlockSpec((1, C_OUT), lambda i: (0, 0)),
            pl.BlockSpec((1, C_OUT), lambda i: (0, 0)),
            pl.BlockSpec((1, C_OUT), lambda i: (0, 0)),
            pl.BlockSpec((C_OUT, C_OUT), lambda i: (0, 0)),
            pl.BlockSpec((1, C_OUT), lambda i: (0, 0)),
        ],
        out_specs=pl.BlockSpec((BN, C_OUT), lambda i: (i, 0)),
        out_shape=jax.ShapeDtypeStruct((n, C_OUT), jnp.float32),
    )(feats, points, w2, b2.reshape(1, -1), gamma.reshape(1, -1),
      beta.reshape(1, -1), w3, b3.reshape(1, -1))


@jax.jit
def kernel(p, x, w1, b1, cw1, cb1, cw2, cb2, w2, b2, gamma, beta, w3, b3):
    b_, c, n = x.shape
    N = b_ * n
    xyz = jnp.transpose(p, (0, 2, 1))
    points = jnp.transpose(x, (0, 2, 1)).reshape(N, C_IN)

    norm_p = (xyz + 1.0) / 2.0
    idx = jnp.clip((norm_p * (G - 1)).astype(jnp.int32), 0, G - 1)
    idx_f = idx.reshape(-1, 3)
    batch_ids = jnp.repeat(jnp.arange(b_, dtype=jnp.int32), n)
    keys = ((batch_ids * G + idx_f[:, 0]) * G + idx_f[:, 1]) * G + idx_f[:, 2]
    offs = jnp.array(
        [(dx, dy, dz) for dx in (-1, 0, 1) for dy in (-1, 0, 1)
         for dz in (-1, 0, 1)], dtype=jnp.int32)
    nbr = idx_f[None, :, :] + offs[:, None, :]
    valid = jnp.all((nbr >= 0) & (nbr <= G - 1), axis=-1)
    nbr_c = jnp.clip(nbr, 0, G - 1)
    qkeys = ((batch_ids[None, :] * G + nbr_c[..., 0]) * G
             + nbr_c[..., 1]) * G + nbr_c[..., 2]
    tab = jnp.full((b_ * G * G * G,), N, jnp.int32)
    tab = tab.at[keys].min(jnp.arange(N, dtype=jnp.int32))
    tv = tab[jnp.where(valid, qkeys, 0)]
    found = valid & (tv < N)
    o_base = (jnp.arange(27, dtype=jnp.int32) * NPAD)[:, None]
    gidx = jnp.where(found, o_base + tv, o_base + N)
    gidx_w = gidx.reshape(27, _NW, N_CHUNKS, CHUNK).transpose(1, 2, 0, 3)

    feats = _lift(points, w1, b1)

    for cw, cb in ((cw1, cb1), (cw2, cb2)):
        featsp = jnp.pad(feats, ((0, NPAD - N), (0, 0)))
        yz = _ymm(featsp, cw).reshape(27 * NPAD, 2 * H)
        feats = _sc_gather_sum(yz, gidx_w, feats, cb)

    out = _final(feats, points, w2, b2, gamma, beta, w3, b3)
    return out.reshape(b_, n, C_OUT)
